# Initial kernel scaffold; baseline (speedup 1.0000x reference)
import jax
import jax.numpy as jnp
from jax.experimental import pallas as pl


def kernel(x, edge_index, seq_batch_node_id, edge_attr, params):
    raise NotImplementedError("write your pallas kernel here")


# speedup vs baseline: 3.5982x; 3.5982x over previous
import functools

import jax
import jax.numpy as jnp
from jax import lax
from jax.experimental import pallas as pl
from jax.experimental.pallas import tpu as pltpu
from jax.experimental.pallas import tpu_sc as plsc

N_NODES = 10000
N_EDGES = 320000
D_FEAT = 128
D_EDGE = 16
H = 128
N_GRAPHS = 16
BN_EPS = 1e-5

NC = 2
NS = 16
NW = NC * NS
EPW = N_EDGES // NW
CH = 80
NCH = EPW // CH
GCH = 25
NG = NCH // GCH

ROWS_A = 624
ROWS_LAST = N_NODES - 15 * ROWS_A


def _sc_edge_body(x_hbm, ee_hbm, src_hbm, dst_hbm, zeros_hbm, agg_hbm,
                  srcv, dstv, xs_v, ee_v, agg_sh, sem):
  c = lax.axis_index("c")
  s = lax.axis_index("s")
  wid = s * NC + c

  @pl.when(s < NS - 1)
  def _():
    pltpu.sync_copy(zeros_hbm.at[pl.ds(s * ROWS_A, ROWS_A)],
                    agg_sh.at[pl.ds(s * ROWS_A, ROWS_A)])

  @pl.when(s == NS - 1)
  def _():
    pltpu.sync_copy(zeros_hbm.at[pl.ds(15 * ROWS_A, ROWS_LAST)],
                    agg_sh.at[pl.ds(15 * ROWS_A, ROWS_LAST)])

  plsc.subcore_barrier()

  def group(g, carry):
    pltpu.sync_copy(src_hbm.at[wid, g], srcv)
    pltpu.sync_copy(dst_hbm.at[wid, g], dstv)

    def chunk(k, carry1):
      base = wid * EPW + (g * GCH + k) * CH
      gather = pltpu.async_copy(x_hbm.at[srcv.at[k]], xs_v, sem)
      pltpu.sync_copy(ee_hbm.at[pl.ds(base, CH)], ee_v)
      gather.wait()

      def edge(e, carry2):
        for j in range(D_FEAT // 16):
          sl = pl.ds(j * 16, 16)
          ee_v[e, sl] = jnp.maximum(ee_v[e, sl] + xs_v[e, sl], 0.0)
        return carry2

      lax.fori_loop(0, CH, edge, 0)
      pltpu.sync_copy(ee_v, agg_sh.at[dstv.at[k]], add=True)
      return carry1

    lax.fori_loop(0, GCH, chunk, 0)
    return carry

  lax.fori_loop(0, NG, group, 0)
  plsc.subcore_barrier()

  @pl.when(s < NS - 1)
  def _():
    pltpu.sync_copy(agg_sh.at[pl.ds(s * ROWS_A, ROWS_A)],
                    agg_hbm.at[c, pl.ds(s * ROWS_A, ROWS_A)])

  @pl.when(s == NS - 1)
  def _():
    pltpu.sync_copy(agg_sh.at[pl.ds(15 * ROWS_A, ROWS_LAST)],
                    agg_hbm.at[c, pl.ds(15 * ROWS_A, ROWS_LAST)])


_sc_edge = pl.kernel(
    _sc_edge_body,
    out_type=jax.ShapeDtypeStruct((NC, N_NODES, D_FEAT), jnp.float32),
    mesh=plsc.VectorSubcoreMesh(core_axis_name="c", subcore_axis_name="s"),
    scratch_types=[
        pltpu.VMEM((GCH, CH), jnp.int32),
        pltpu.VMEM((GCH, CH), jnp.int32),
        pltpu.VMEM((CH, D_FEAT), jnp.float32),
        pltpu.VMEM((CH, D_FEAT), jnp.float32),
        pltpu.VMEM_SHARED((N_NODES, D_FEAT), jnp.float32),
        pltpu.SemaphoreType.DMA,
    ],
)


def _ee_block(ea_ref, we_ref, be_ref, out_ref):
  out_ref[...] = (
      jnp.dot(ea_ref[...], we_ref[...], preferred_element_type=jnp.float32)
      + be_ref[...])


_EE_B = 2000

_ee_call = pl.pallas_call(
    _ee_block,
    grid=(N_EDGES // _EE_B,),
    in_specs=[
        pl.BlockSpec((_EE_B, D_EDGE), lambda b: (b, 0)),
        pl.BlockSpec((D_EDGE, H), lambda b: (0, 0)),
        pl.BlockSpec((1, H), lambda b: (0, 0)),
    ],
    out_specs=pl.BlockSpec((_EE_B, H), lambda b: (b, 0)),
    out_shape=jax.ShapeDtypeStruct((N_EDGES, H), jnp.float32),
)


def _node_block(x_ref, agg_ref, w1, b1, g1, t1, w2, b2, g2, t2, bid_ref,
                ne_ref, pool_ref):
  h = x_ref[...] + agg_ref[0] + agg_ref[1]
  h = jnp.dot(h, w1[...], preferred_element_type=jnp.float32) + b1[...]
  m = jnp.mean(h, axis=0, keepdims=True)
  v = jnp.mean(jnp.square(h - m), axis=0, keepdims=True)
  h = (h - m) / jnp.sqrt(v + BN_EPS) * g1[...] + t1[...]
  h = jnp.maximum(h, 0.0)
  h = jnp.dot(h, w2[...], preferred_element_type=jnp.float32) + b2[...]
  m = jnp.mean(h, axis=0, keepdims=True)
  v = jnp.mean(jnp.square(h - m), axis=0, keepdims=True)
  h = (h - m) / jnp.sqrt(v + BN_EPS) * g2[...] + t2[...]
  h = jnp.maximum(h, 0.0)
  ne_ref[...] = h
  onehot_t = (bid_ref[...] ==
              lax.broadcasted_iota(jnp.int32, (N_GRAPHS, N_NODES), 0)
              ).astype(jnp.float32)
  pool_ref[...] = jnp.dot(onehot_t, h, preferred_element_type=jnp.float32)


_node_call = pl.pallas_call(
    _node_block,
    out_shape=(
        jax.ShapeDtypeStruct((N_NODES, H), jnp.float32),
        jax.ShapeDtypeStruct((N_GRAPHS, H), jnp.float32),
    ),
)


def kernel(x, edge_index, seq_batch_node_id, edge_attr, params):
  src = edge_index[0].astype(jnp.int32).reshape(NW, NG, GCH, CH)
  dst = edge_index[1].astype(jnp.int32).reshape(NW, NG, GCH, CH)
  zeros = jnp.zeros((N_NODES, D_FEAT), jnp.float32)
  bid = seq_batch_node_id.astype(jnp.int32).reshape(1, N_NODES)

  pools = []
  h = x
  for i in range(len(params)):
    p = params[i]
    ee = _ee_call(edge_attr, p['We'], p['be'].reshape(1, H))
    agg = _sc_edge(h, ee, src, dst, zeros)
    h, pool = _node_call(
        h, agg,
        p['W1'], p['b1'].reshape(1, H), p['g1'].reshape(1, H),
        p['bt1'].reshape(1, H),
        p['W2'], p['b2'].reshape(1, H), p['g2'].reshape(1, H),
        p['bt2'].reshape(1, H),
        bid)
    pools.append(pool)
  return jnp.concatenate(pools, axis=1)
